# Initial kernel scaffold; baseline (speedup 1.0000x reference)
import functools

import jax
import jax.numpy as jnp
from jax import lax
from jax.experimental import pallas as pl
from jax.experimental.pallas import tpu as pltpu
from jax.experimental.pallas import tpu_sc as plsc

N_NODES = 50000
N_PAD = 51200
E = 800000
ER = E // 128
G = 256
NC, NS = 2, 16
ROWS_PS = ER // NS
SLICE = N_PAD // NS
BLK = 2048
GRID = N_PAD // BLK

_mesh = plsc.VectorSubcoreMesh(core_axis_name="c", subcore_axis_name="s",
                               num_cores=NC, num_subcores=NS)


DEG_ROWS = ER // 32
DEG_TAIL = ER - 32 * DEG_ROWS


@functools.partial(
    pl.kernel,
    mesh=_mesh,
    out_type=jax.ShapeDtypeStruct((32, N_PAD), jnp.float32),
    scratch_types=[
        pltpu.VMEM((DEG_ROWS, 128), jnp.int32),
        pltpu.VMEM((DEG_TAIL, 128), jnp.int32),
        pltpu.VMEM((N_PAD,), jnp.float32),
    ],
)
def _deg_kernel(dst2d_hbm, out_hbm, stage, tailb, hist):
    c = lax.axis_index("c")
    s = lax.axis_index("s")
    wid = s * NC + c

    zero16 = jnp.zeros((16,), jnp.float32)
    ones16 = jnp.ones((16,), jnp.float32)

    def zb(i, _):
        hist[pl.ds(i * 16, 16)] = zero16
        return 0
    lax.fori_loop(0, N_PAD // 16, zb, 0)

    pltpu.sync_copy(dst2d_hbm.at[pl.ds(wid * DEG_ROWS, DEG_ROWS)], stage)
    pltpu.sync_copy(dst2d_hbm.at[pl.ds(32 * DEG_ROWS, DEG_TAIL)], tailb)

    def rowloop(r, _):
        def kloop(k, _):
            idx = stage[r, pl.ds(k * 16, 16)]
            plsc.addupdate_scatter(hist, [idx], ones16)
            return 0
        return lax.fori_loop(0, 8, kloop, 0)
    lax.fori_loop(0, DEG_ROWS, rowloop, 0)

    @pl.when(wid < DEG_TAIL)
    def _():
        def kloop(k, _):
            idx = tailb[wid, pl.ds(k * 16, 16)]
            plsc.addupdate_scatter(hist, [idx], ones16)
            return 0
        lax.fori_loop(0, 8, kloop, 0)

    pltpu.sync_copy(hist, out_hbm.at[wid])


@functools.partial(
    pl.kernel,
    mesh=_mesh,
    out_type=(jax.ShapeDtypeStruct((N_PAD, 32), jnp.float32),
              jax.ShapeDtypeStruct((N_PAD, 32), jnp.float32)),
    scratch_types=[
        pltpu.VMEM_SHARED((N_PAD, 32), jnp.float32),
        pltpu.VMEM((2, 128), jnp.int32),
        pltpu.VMEM((128, 32), jnp.float32),
    ],
)
def _conv_kernel(y0_hbm, y1_hbm, sd_hbm, zeros_hbm, a0_hbm, a1_hbm,
                 accsh, idxb, rowb):
    c = lax.axis_index("c")
    s = lax.axis_index("s")

    pltpu.sync_copy(zeros_hbm, accsh.at[pl.ds(s * SLICE, SLICE)])
    plsc.subcore_barrier()

    rem = ER - NS * ROWS_PS
    base = s * ROWS_PS + jnp.minimum(s, rem)
    nrows = ROWS_PS + jnp.where(s < rem, 1, 0)

    def run(yc, ac):
        def edge(i, _):
            row = base + i
            pltpu.sync_copy(sd_hbm.at[row], idxb)
            pltpu.sync_copy(yc.at[idxb.at[0]], rowb)
            pltpu.sync_copy(rowb, accsh.at[idxb.at[1]], add=True)
            return 0
        lax.fori_loop(0, nrows, edge, 0)
        plsc.subcore_barrier()
        pltpu.sync_copy(accsh.at[pl.ds(s * SLICE, SLICE)],
                        ac.at[pl.ds(s * SLICE, SLICE)])

    @pl.when(c == 0)
    def _():
        run(y0_hbm, a0_hbm)

    @pl.when(c == 1)
    def _():
        run(y1_hbm, a1_hbm)


def _embed_body(x_ref, degp_ref, emb_ref, dep_ref, pw_ref, pb_ref, w1_ref,
                y0_ref, y1_ref, dinv_ref):
    xi = x_ref[...]
    comb0 = xi[:, 0]
    comb1 = jnp.clip(xi[:, 1], 0, 19)
    t1 = jnp.concatenate(
        [emb_ref[...] @ pw_ref[:16, :], jnp.zeros((7, 64), jnp.float32)], axis=0)
    t2 = jnp.concatenate(
        [dep_ref[...] @ pw_ref[16:, :], jnp.zeros((12, 64), jnp.float32)], axis=0)
    oh0 = (lax.broadcasted_iota(jnp.int32, (BLK, 16), 1)
           == comb0[:, None]).astype(jnp.float32)
    oh1 = (lax.broadcasted_iota(jnp.int32, (BLK, 32), 1)
           == comb1[:, None]).astype(jnp.float32)
    h0 = (oh0 @ t1) + (oh1 @ t2) + pb_ref[...][None, :]
    xw1 = h0 @ w1_ref[...]
    deg = jnp.sum(degp_ref[...], axis=0)
    dinv = lax.rsqrt(deg + 1.0)
    y = xw1 * dinv[:, None]
    y0_ref[...] = y[:, :32]
    y1_ref[...] = y[:, 32:]
    dinv_ref[...] = dinv


def _embed_call(x_pad, deg_parts, emb_table, depth_table, proj_W, proj_b, g1_W):
    return pl.pallas_call(
        _embed_body,
        grid=(GRID,),
        in_specs=[
            pl.BlockSpec((BLK, 2), lambda i: (i, 0)),
            pl.BlockSpec((32, BLK), lambda i: (0, i)),
            pl.BlockSpec((9, 16), lambda i: (0, 0)),
            pl.BlockSpec((20, 16), lambda i: (0, 0)),
            pl.BlockSpec((32, 64), lambda i: (0, 0)),
            pl.BlockSpec((64,), lambda i: (0,)),
            pl.BlockSpec((64, 64), lambda i: (0, 0)),
        ],
        out_specs=[
            pl.BlockSpec((BLK, 32), lambda i: (i, 0)),
            pl.BlockSpec((BLK, 32), lambda i: (i, 0)),
            pl.BlockSpec((BLK,), lambda i: (i,)),
        ],
        out_shape=[
            jax.ShapeDtypeStruct((N_PAD, 32), jnp.float32),
            jax.ShapeDtypeStruct((N_PAD, 32), jnp.float32),
            jax.ShapeDtypeStruct((N_PAD,), jnp.float32),
        ],
    )(x_pad, deg_parts, emb_table, depth_table, proj_W, proj_b, g1_W)


def _post1_body(a0_ref, a1_ref, y0_ref, y1_ref, dinv_ref, b_ref, w_ref,
                z0_ref, z1_ref):
    dinv = dinv_ref[...]
    h = jnp.concatenate([a0_ref[...] + y0_ref[...],
                         a1_ref[...] + y1_ref[...]], axis=1)
    h = jnp.maximum(h * dinv[:, None] + b_ref[...][None, :], 0.0)
    y2 = (h @ w_ref[...]) * dinv[:, None]
    z0_ref[...] = y2[:, :32]
    z1_ref[...] = y2[:, 32:]


def _post1_call(a0, a1, y0, y1, dinv, b1, g2_W):
    return pl.pallas_call(
        _post1_body,
        grid=(GRID,),
        in_specs=[
            pl.BlockSpec((BLK, 32), lambda i: (i, 0)),
            pl.BlockSpec((BLK, 32), lambda i: (i, 0)),
            pl.BlockSpec((BLK, 32), lambda i: (i, 0)),
            pl.BlockSpec((BLK, 32), lambda i: (i, 0)),
            pl.BlockSpec((BLK,), lambda i: (i,)),
            pl.BlockSpec((64,), lambda i: (0,)),
            pl.BlockSpec((64, 64), lambda i: (0, 0)),
        ],
        out_specs=[
            pl.BlockSpec((BLK, 32), lambda i: (i, 0)),
            pl.BlockSpec((BLK, 32), lambda i: (i, 0)),
        ],
        out_shape=[
            jax.ShapeDtypeStruct((N_PAD, 32), jnp.float32),
            jax.ShapeDtypeStruct((N_PAD, 32), jnp.float32),
        ],
    )(a0, a1, y0, y1, dinv, b1, g2_W)


def _post2_body(a0_ref, a1_ref, y0_ref, y1_ref, dinv_ref, b_ref, batch_ref,
                fw_ref, fb_ref, out_ref, sacc, cacc):
    i = pl.program_id(0)

    @pl.when(i == 0)
    def _():
        sacc[...] = jnp.zeros_like(sacc)
        cacc[...] = jnp.zeros_like(cacc)

    dinv = dinv_ref[...]
    h = jnp.concatenate([a0_ref[...] + y0_ref[...],
                         a1_ref[...] + y1_ref[...]], axis=1)
    h = jnp.maximum(h * dinv[:, None] + b_ref[...][None, :], 0.0)
    ohT = (lax.broadcasted_iota(jnp.int32, (G, BLK), 0)
           == batch_ref[...][None, :]).astype(jnp.float32)
    sacc[...] += ohT @ h
    cacc[...] += jnp.sum(ohT, axis=1)
    pooled = sacc[...] / jnp.maximum(cacc[...], 1.0)[:, None]
    out_ref[...] = pooled @ fw_ref[...] + fb_ref[...][None, :]


def _post2_call(a0, a1, y0, y1, dinv, b2, batch_pad, final_W, final_b):
    return pl.pallas_call(
        _post2_body,
        grid=(GRID,),
        in_specs=[
            pl.BlockSpec((BLK, 32), lambda i: (i, 0)),
            pl.BlockSpec((BLK, 32), lambda i: (i, 0)),
            pl.BlockSpec((BLK, 32), lambda i: (i, 0)),
            pl.BlockSpec((BLK, 32), lambda i: (i, 0)),
            pl.BlockSpec((BLK,), lambda i: (i,)),
            pl.BlockSpec((64,), lambda i: (0,)),
            pl.BlockSpec((BLK,), lambda i: (i,)),
            pl.BlockSpec((64, 128), lambda i: (0, 0)),
            pl.BlockSpec((128,), lambda i: (0,)),
        ],
        out_specs=pl.BlockSpec((G, 128), lambda i: (0, 0)),
        out_shape=jax.ShapeDtypeStruct((G, 128), jnp.float32),
        scratch_shapes=[
            pltpu.VMEM((G, 64), jnp.float32),
            pltpu.VMEM((G,), jnp.float32),
        ],
    )(a0, a1, y0, y1, dinv, b2, batch_pad, final_W, final_b)


def kernel(x, edge_index, batch, emb_table, depth_table, proj_W, proj_b,
           g1_W, g1_b, g2_W, g2_b, final_W, final_b):
    x_pad = jnp.pad(x, ((0, N_PAD - N_NODES), (0, 0)))
    batch_pad = jnp.pad(batch, (0, N_PAD - N_NODES), constant_values=G)
    src2d = edge_index[0].reshape(ER, 128)
    dst2d = edge_index[1].reshape(ER, 128)
    sd = jnp.stack([src2d, dst2d], axis=1)
    zeros_sl = jnp.zeros((SLICE, 32), jnp.float32)

    deg_parts = _deg_kernel(dst2d)
    y0, y1, dinv = _embed_call(x_pad, deg_parts, emb_table, depth_table,
                               proj_W, proj_b, g1_W)
    a0, a1 = _conv_kernel(y0, y1, sd, zeros_sl)
    z0, z1 = _post1_call(a0, a1, y0, y1, dinv, g1_b, g2_W)
    c0, c1 = _conv_kernel(z0, z1, sd, zeros_sl)
    return _post2_call(c0, c1, z0, z1, dinv, g2_b, batch_pad, final_W, final_b)


# speedup vs baseline: 18.9662x; 18.9662x over previous
import functools

import jax
import jax.numpy as jnp
from jax import lax
from jax.experimental import pallas as pl
from jax.experimental.pallas import tpu as pltpu
from jax.experimental.pallas import tpu_sc as plsc

N_NODES = 50000
N_PAD = 51200
E = 800000
ER = 6272
E_PAD = ER * 128
G = 256
NC, NS = 2, 16
GRP_PS = ER // 8 // NS
SLICE = N_PAD // NS
BLK = 2048
GRID = N_PAD // BLK

def _mesh():
    return plsc.VectorSubcoreMesh(core_axis_name="c", subcore_axis_name="s",
                                  num_cores=NC, num_subcores=NS)


DEG_CHUNK = E_PAD // 32


def _deg_body(dst_hbm, out_hbm, stage, hist):
    c = lax.axis_index("c")
    s = lax.axis_index("s")
    wid = s * NC + c

    zero16 = jnp.zeros((16,), jnp.float32)
    ones16 = jnp.ones((16,), jnp.float32)

    def zb(i, _):
        hist[0, pl.ds(i * 16, 16)] = zero16
        return 0
    lax.fori_loop(0, N_PAD // 16, zb, 0)

    pltpu.sync_copy(dst_hbm.at[pl.ds(wid * DEG_CHUNK, DEG_CHUNK)], stage)

    def kloop(k, _):
        idx = stage[pl.ds(k * 16, 16)]
        plsc.addupdate_scatter(hist.at[0], [idx], ones16)
        return 0
    lax.fori_loop(0, DEG_CHUNK // 16, kloop, 0)

    pltpu.sync_copy(hist, out_hbm.at[wid])


@functools.lru_cache(maxsize=None)
def _deg_kernel_fn():
    return pl.kernel(
        _deg_body,
        mesh=_mesh(),
        out_type=jax.ShapeDtypeStruct((32, 1, N_PAD), jnp.float32),
        scratch_types=[
            pltpu.VMEM((DEG_CHUNK,), jnp.int32),
            pltpu.VMEM((1, N_PAD), jnp.float32),
        ],
        compiler_params=pltpu.CompilerParams(needs_layout_passes=False, use_tc_tiling_on_sc=False),
    )


def _deg_kernel(dst_flat):
    return _deg_kernel_fn()(dst_flat)


def _conv_body(y0_hbm, y1_hbm, src_hbm, dst_hbm, zeros_hbm, a0_hbm, a1_hbm,
               accsh, srcg, dstg, rowb):
    c = lax.axis_index("c")
    s = lax.axis_index("s")

    pltpu.sync_copy(zeros_hbm, accsh.at[pl.ds(s * SLICE, SLICE)])
    plsc.subcore_barrier()

    def run(yc, ac):
        def grp(g, _):
            row0 = (s * GRP_PS + g) * 8
            pltpu.sync_copy(src_hbm.at[pl.ds(row0, 8)], srcg)
            pltpu.sync_copy(dst_hbm.at[pl.ds(row0, 8)], dstg)
            for j in range(8):
                pltpu.sync_copy(yc.at[srcg.at[j]], rowb)
                pltpu.sync_copy(rowb, accsh.at[dstg.at[j]], add=True)
            return 0
        lax.fori_loop(0, GRP_PS, grp, 0)
        plsc.subcore_barrier()
        pltpu.sync_copy(accsh.at[pl.ds(s * SLICE, SLICE)],
                        ac.at[pl.ds(s * SLICE, SLICE)])

    @pl.when(c == 0)
    def _():
        run(y0_hbm, a0_hbm)

    @pl.when(c == 1)
    def _():
        run(y1_hbm, a1_hbm)


@functools.lru_cache(maxsize=None)
def _conv_kernel_fn():
    return pl.kernel(
        _conv_body,
        mesh=_mesh(),
        out_type=(jax.ShapeDtypeStruct((N_PAD, 32), jnp.float32),
                  jax.ShapeDtypeStruct((N_PAD, 32), jnp.float32)),
        scratch_types=[
            pltpu.VMEM_SHARED((N_PAD, 32), jnp.float32),
            pltpu.VMEM((8, 128), jnp.int32),
            pltpu.VMEM((8, 128), jnp.int32),
            pltpu.VMEM((128, 32), jnp.float32),
        ],
        compiler_params=pltpu.CompilerParams(needs_layout_passes=False, use_tc_tiling_on_sc=False),
    )


def _conv_kernel(y0, y1, src2d, dst2d, zeros_sl):
    return _conv_kernel_fn()(y0, y1, src2d, dst2d, zeros_sl)


def _embed_body(x_ref, degp_ref, emb_ref, dep_ref, pw_ref, pb_ref, w1_ref,
                y0_ref, y1_ref, dinv_ref):
    xi = x_ref[...]
    comb0 = xi[:, 0]
    comb1 = jnp.clip(xi[:, 1], 0, 19)
    t1 = jnp.concatenate(
        [emb_ref[...] @ pw_ref[:16, :], jnp.zeros((7, 64), jnp.float32)], axis=0)
    t2 = jnp.concatenate(
        [dep_ref[...] @ pw_ref[16:, :], jnp.zeros((12, 64), jnp.float32)], axis=0)
    oh0 = (lax.broadcasted_iota(jnp.int32, (BLK, 16), 1)
           == comb0[:, None]).astype(jnp.float32)
    oh1 = (lax.broadcasted_iota(jnp.int32, (BLK, 32), 1)
           == comb1[:, None]).astype(jnp.float32)
    h0 = (oh0 @ t1) + (oh1 @ t2) + pb_ref[...][None, :]
    xw1 = h0 @ w1_ref[...]
    deg = jnp.sum(degp_ref[...], axis=(0, 1))
    dinv = lax.rsqrt(deg + 1.0)
    y = xw1 * dinv[:, None]
    y0_ref[...] = y[:, :32]
    y1_ref[...] = y[:, 32:]
    dinv_ref[...] = dinv


def _embed_call(x_pad, deg_parts, emb_table, depth_table, proj_W, proj_b, g1_W):
    return pl.pallas_call(
        _embed_body,
        grid=(GRID,),
        in_specs=[
            pl.BlockSpec((BLK, 2), lambda i: (i, 0)),
            pl.BlockSpec((32, 1, BLK), lambda i: (0, 0, i)),
            pl.BlockSpec((9, 16), lambda i: (0, 0)),
            pl.BlockSpec((20, 16), lambda i: (0, 0)),
            pl.BlockSpec((32, 64), lambda i: (0, 0)),
            pl.BlockSpec((64,), lambda i: (0,)),
            pl.BlockSpec((64, 64), lambda i: (0, 0)),
        ],
        out_specs=[
            pl.BlockSpec((BLK, 32), lambda i: (i, 0)),
            pl.BlockSpec((BLK, 32), lambda i: (i, 0)),
            pl.BlockSpec((BLK,), lambda i: (i,)),
        ],
        out_shape=[
            jax.ShapeDtypeStruct((N_PAD, 32), jnp.float32),
            jax.ShapeDtypeStruct((N_PAD, 32), jnp.float32),
            jax.ShapeDtypeStruct((N_PAD,), jnp.float32),
        ],
    )(x_pad, deg_parts, emb_table, depth_table, proj_W, proj_b, g1_W)


def _post1_body(a0_ref, a1_ref, y0_ref, y1_ref, dinv_ref, b_ref, w_ref,
                z0_ref, z1_ref):
    dinv = dinv_ref[...]
    h = jnp.concatenate([a0_ref[...] + y0_ref[...],
                         a1_ref[...] + y1_ref[...]], axis=1)
    h = jnp.maximum(h * dinv[:, None] + b_ref[...][None, :], 0.0)
    y2 = (h @ w_ref[...]) * dinv[:, None]
    z0_ref[...] = y2[:, :32]
    z1_ref[...] = y2[:, 32:]


def _post1_call(a0, a1, y0, y1, dinv, b1, g2_W):
    return pl.pallas_call(
        _post1_body,
        grid=(GRID,),
        in_specs=[
            pl.BlockSpec((BLK, 32), lambda i: (i, 0)),
            pl.BlockSpec((BLK, 32), lambda i: (i, 0)),
            pl.BlockSpec((BLK, 32), lambda i: (i, 0)),
            pl.BlockSpec((BLK, 32), lambda i: (i, 0)),
            pl.BlockSpec((BLK,), lambda i: (i,)),
            pl.BlockSpec((64,), lambda i: (0,)),
            pl.BlockSpec((64, 64), lambda i: (0, 0)),
        ],
        out_specs=[
            pl.BlockSpec((BLK, 32), lambda i: (i, 0)),
            pl.BlockSpec((BLK, 32), lambda i: (i, 0)),
        ],
        out_shape=[
            jax.ShapeDtypeStruct((N_PAD, 32), jnp.float32),
            jax.ShapeDtypeStruct((N_PAD, 32), jnp.float32),
        ],
    )(a0, a1, y0, y1, dinv, b1, g2_W)


def _post2_body(a0_ref, a1_ref, y0_ref, y1_ref, dinv_ref, b_ref, batch_ref,
                fw_ref, fb_ref, out_ref, sacc, cacc):
    i = pl.program_id(0)

    @pl.when(i == 0)
    def _():
        sacc[...] = jnp.zeros_like(sacc)
        cacc[...] = jnp.zeros_like(cacc)

    dinv = dinv_ref[...]
    h = jnp.concatenate([a0_ref[...] + y0_ref[...],
                         a1_ref[...] + y1_ref[...]], axis=1)
    h = jnp.maximum(h * dinv[:, None] + b_ref[...][None, :], 0.0)
    ohT = (lax.broadcasted_iota(jnp.int32, (G, BLK), 0)
           == batch_ref[...][None, :]).astype(jnp.float32)
    sacc[...] += ohT @ h
    cacc[...] += jnp.sum(ohT, axis=1)
    pooled = sacc[...] / jnp.maximum(cacc[...], 1.0)[:, None]
    out_ref[...] = pooled @ fw_ref[...] + fb_ref[...][None, :]


def _post2_call(a0, a1, y0, y1, dinv, b2, batch_pad, final_W, final_b):
    return pl.pallas_call(
        _post2_body,
        grid=(GRID,),
        in_specs=[
            pl.BlockSpec((BLK, 32), lambda i: (i, 0)),
            pl.BlockSpec((BLK, 32), lambda i: (i, 0)),
            pl.BlockSpec((BLK, 32), lambda i: (i, 0)),
            pl.BlockSpec((BLK, 32), lambda i: (i, 0)),
            pl.BlockSpec((BLK,), lambda i: (i,)),
            pl.BlockSpec((64,), lambda i: (0,)),
            pl.BlockSpec((BLK,), lambda i: (i,)),
            pl.BlockSpec((64, 128), lambda i: (0, 0)),
            pl.BlockSpec((128,), lambda i: (0,)),
        ],
        out_specs=pl.BlockSpec((G, 128), lambda i: (0, 0)),
        out_shape=jax.ShapeDtypeStruct((G, 128), jnp.float32),
        scratch_shapes=[
            pltpu.VMEM((G, 64), jnp.float32),
            pltpu.VMEM((G,), jnp.float32),
        ],
    )(a0, a1, y0, y1, dinv, b2, batch_pad, final_W, final_b)


def kernel(x, edge_index, batch, emb_table, depth_table, proj_W, proj_b,
           g1_W, g1_b, g2_W, g2_b, final_W, final_b):
    x_pad = jnp.pad(x, ((0, N_PAD - N_NODES), (0, 0)))
    batch_pad = jnp.pad(batch, (0, N_PAD - N_NODES), constant_values=G)
    src_flat = jnp.pad(edge_index[0], (0, E_PAD - E), constant_values=N_NODES)
    dst_flat = jnp.pad(edge_index[1], (0, E_PAD - E), constant_values=N_NODES)
    src2d = src_flat.reshape(ER, 128)
    dst2d = dst_flat.reshape(ER, 128)
    zeros_sl = jnp.zeros((SLICE, 32), jnp.float32)

    deg_parts = _deg_kernel(dst_flat)
    y0, y1, dinv = _embed_call(x_pad, deg_parts, emb_table, depth_table,
                               proj_W, proj_b, g1_W)
    a0, a1 = _conv_kernel(y0, y1, src2d, dst2d, zeros_sl)
    z0, z1 = _post1_call(a0, a1, y0, y1, dinv, g1_b, g2_W)
    c0, c1 = _conv_kernel(z0, z1, src2d, dst2d, zeros_sl)
    return _post2_call(c0, c1, z0, z1, dinv, g2_b, batch_pad, final_W, final_b)


# speedup vs baseline: 24.0839x; 1.2698x over previous
import functools

import jax
import jax.numpy as jnp
from jax import lax
from jax.experimental import pallas as pl
from jax.experimental.pallas import tpu as pltpu
from jax.experimental.pallas import tpu_sc as plsc

N_NODES = 50000
N_PAD = 51200
E = 800000
ER = 6272
E_PAD = ER * 128
G = 256
NC, NS = 2, 16
EG = 512
GRP_PS = E_PAD // NS // EG
SLICE = N_PAD // NS
BLK = 2048
GRID = N_PAD // BLK

def _mesh():
    return plsc.VectorSubcoreMesh(core_axis_name="c", subcore_axis_name="s",
                                  num_cores=NC, num_subcores=NS)


DEG_CHUNK = E_PAD // 32


def _deg_body(dst_hbm, out_hbm, stage, hist):
    c = lax.axis_index("c")
    s = lax.axis_index("s")
    wid = s * NC + c

    zero16 = jnp.zeros((16,), jnp.float32)
    ones16 = jnp.ones((16,), jnp.float32)

    def zb(i, _):
        hist[0, pl.ds(i * 16, 16)] = zero16
        return 0
    lax.fori_loop(0, N_PAD // 16, zb, 0)

    pltpu.sync_copy(dst_hbm.at[pl.ds(wid * DEG_CHUNK, DEG_CHUNK)], stage)

    def kloop(k, _):
        idx = stage[pl.ds(k * 16, 16)]
        plsc.addupdate_scatter(hist.at[0], [idx], ones16)
        return 0
    lax.fori_loop(0, DEG_CHUNK // 16, kloop, 0)

    pltpu.sync_copy(hist, out_hbm.at[wid])


@functools.lru_cache(maxsize=None)
def _deg_kernel_fn():
    return pl.kernel(
        _deg_body,
        mesh=_mesh(),
        out_type=jax.ShapeDtypeStruct((32, 1, N_PAD), jnp.float32),
        scratch_types=[
            pltpu.VMEM((DEG_CHUNK,), jnp.int32),
            pltpu.VMEM((1, N_PAD), jnp.float32),
        ],
        compiler_params=pltpu.CompilerParams(needs_layout_passes=False, use_tc_tiling_on_sc=False),
    )


def _deg_kernel(dst_flat):
    return _deg_kernel_fn()(dst_flat)


def _conv_body(y0_hbm, y1_hbm, src_hbm, dst_hbm, zeros_hbm, a0_hbm, a1_hbm,
               accsh, srcg, dstg, rowb):
    c = lax.axis_index("c")
    s = lax.axis_index("s")

    pltpu.sync_copy(zeros_hbm, accsh.at[pl.ds(s * SLICE, SLICE)])
    plsc.subcore_barrier()

    def run(yc, ac):
        def grp(g, _):
            e0 = (s * GRP_PS + g) * EG
            pltpu.sync_copy(src_hbm.at[pl.ds(e0, EG)], srcg)
            pltpu.sync_copy(dst_hbm.at[pl.ds(e0, EG)], dstg)
            pltpu.sync_copy(yc.at[srcg], rowb)
            pltpu.sync_copy(rowb, accsh.at[dstg], add=True)
            return 0
        lax.fori_loop(0, GRP_PS, grp, 0)
        plsc.subcore_barrier()
        pltpu.sync_copy(accsh.at[pl.ds(s * SLICE, SLICE)],
                        ac.at[pl.ds(s * SLICE, SLICE)])

    @pl.when(c == 0)
    def _():
        run(y0_hbm, a0_hbm)

    @pl.when(c == 1)
    def _():
        run(y1_hbm, a1_hbm)


@functools.lru_cache(maxsize=None)
def _conv_kernel_fn():
    return pl.kernel(
        _conv_body,
        mesh=_mesh(),
        out_type=(jax.ShapeDtypeStruct((N_PAD, 32), jnp.float32),
                  jax.ShapeDtypeStruct((N_PAD, 32), jnp.float32)),
        scratch_types=[
            pltpu.VMEM_SHARED((N_PAD, 32), jnp.float32),
            pltpu.VMEM((EG,), jnp.int32),
            pltpu.VMEM((EG,), jnp.int32),
            pltpu.VMEM((EG, 32), jnp.float32),
        ],
        compiler_params=pltpu.CompilerParams(needs_layout_passes=False, use_tc_tiling_on_sc=False),
    )


def _conv_kernel(y0, y1, src_flat, dst_flat, zeros_sl):
    return _conv_kernel_fn()(y0, y1, src_flat, dst_flat, zeros_sl)


def _embed_body(x_ref, degp_ref, emb_ref, dep_ref, pw_ref, pb_ref, w1_ref,
                y0_ref, y1_ref, dinv_ref):
    xi = x_ref[...]
    comb0 = xi[:, 0]
    comb1 = jnp.clip(xi[:, 1], 0, 19)
    t1 = jnp.concatenate(
        [emb_ref[...] @ pw_ref[:16, :], jnp.zeros((7, 64), jnp.float32)], axis=0)
    t2 = jnp.concatenate(
        [dep_ref[...] @ pw_ref[16:, :], jnp.zeros((12, 64), jnp.float32)], axis=0)
    oh0 = (lax.broadcasted_iota(jnp.int32, (BLK, 16), 1)
           == comb0[:, None]).astype(jnp.float32)
    oh1 = (lax.broadcasted_iota(jnp.int32, (BLK, 32), 1)
           == comb1[:, None]).astype(jnp.float32)
    h0 = (oh0 @ t1) + (oh1 @ t2) + pb_ref[...][None, :]
    xw1 = h0 @ w1_ref[...]
    deg = jnp.sum(degp_ref[...], axis=(0, 1))
    dinv = lax.rsqrt(deg + 1.0)
    y = xw1 * dinv[:, None]
    y0_ref[...] = y[:, :32]
    y1_ref[...] = y[:, 32:]
    dinv_ref[...] = dinv


def _embed_call(x_pad, deg_parts, emb_table, depth_table, proj_W, proj_b, g1_W):
    return pl.pallas_call(
        _embed_body,
        grid=(GRID,),
        in_specs=[
            pl.BlockSpec((BLK, 2), lambda i: (i, 0)),
            pl.BlockSpec((32, 1, BLK), lambda i: (0, 0, i)),
            pl.BlockSpec((9, 16), lambda i: (0, 0)),
            pl.BlockSpec((20, 16), lambda i: (0, 0)),
            pl.BlockSpec((32, 64), lambda i: (0, 0)),
            pl.BlockSpec((64,), lambda i: (0,)),
            pl.BlockSpec((64, 64), lambda i: (0, 0)),
        ],
        out_specs=[
            pl.BlockSpec((BLK, 32), lambda i: (i, 0)),
            pl.BlockSpec((BLK, 32), lambda i: (i, 0)),
            pl.BlockSpec((BLK,), lambda i: (i,)),
        ],
        out_shape=[
            jax.ShapeDtypeStruct((N_PAD, 32), jnp.float32),
            jax.ShapeDtypeStruct((N_PAD, 32), jnp.float32),
            jax.ShapeDtypeStruct((N_PAD,), jnp.float32),
        ],
    )(x_pad, deg_parts, emb_table, depth_table, proj_W, proj_b, g1_W)


def _post1_body(a0_ref, a1_ref, y0_ref, y1_ref, dinv_ref, b_ref, w_ref,
                z0_ref, z1_ref):
    dinv = dinv_ref[...]
    h = jnp.concatenate([a0_ref[...] + y0_ref[...],
                         a1_ref[...] + y1_ref[...]], axis=1)
    h = jnp.maximum(h * dinv[:, None] + b_ref[...][None, :], 0.0)
    y2 = (h @ w_ref[...]) * dinv[:, None]
    z0_ref[...] = y2[:, :32]
    z1_ref[...] = y2[:, 32:]


def _post1_call(a0, a1, y0, y1, dinv, b1, g2_W):
    return pl.pallas_call(
        _post1_body,
        grid=(GRID,),
        in_specs=[
            pl.BlockSpec((BLK, 32), lambda i: (i, 0)),
            pl.BlockSpec((BLK, 32), lambda i: (i, 0)),
            pl.BlockSpec((BLK, 32), lambda i: (i, 0)),
            pl.BlockSpec((BLK, 32), lambda i: (i, 0)),
            pl.BlockSpec((BLK,), lambda i: (i,)),
            pl.BlockSpec((64,), lambda i: (0,)),
            pl.BlockSpec((64, 64), lambda i: (0, 0)),
        ],
        out_specs=[
            pl.BlockSpec((BLK, 32), lambda i: (i, 0)),
            pl.BlockSpec((BLK, 32), lambda i: (i, 0)),
        ],
        out_shape=[
            jax.ShapeDtypeStruct((N_PAD, 32), jnp.float32),
            jax.ShapeDtypeStruct((N_PAD, 32), jnp.float32),
        ],
    )(a0, a1, y0, y1, dinv, b1, g2_W)


def _post2_body(a0_ref, a1_ref, y0_ref, y1_ref, dinv_ref, b_ref, batch_ref,
                fw_ref, fb_ref, out_ref, sacc, cacc):
    i = pl.program_id(0)

    @pl.when(i == 0)
    def _():
        sacc[...] = jnp.zeros_like(sacc)
        cacc[...] = jnp.zeros_like(cacc)

    dinv = dinv_ref[...]
    h = jnp.concatenate([a0_ref[...] + y0_ref[...],
                         a1_ref[...] + y1_ref[...]], axis=1)
    h = jnp.maximum(h * dinv[:, None] + b_ref[...][None, :], 0.0)
    ohT = (lax.broadcasted_iota(jnp.int32, (G, BLK), 0)
           == batch_ref[...][None, :]).astype(jnp.float32)
    sacc[...] += ohT @ h
    cacc[...] += jnp.sum(ohT, axis=1)
    pooled = sacc[...] / jnp.maximum(cacc[...], 1.0)[:, None]
    out_ref[...] = pooled @ fw_ref[...] + fb_ref[...][None, :]


def _post2_call(a0, a1, y0, y1, dinv, b2, batch_pad, final_W, final_b):
    return pl.pallas_call(
        _post2_body,
        grid=(GRID,),
        in_specs=[
            pl.BlockSpec((BLK, 32), lambda i: (i, 0)),
            pl.BlockSpec((BLK, 32), lambda i: (i, 0)),
            pl.BlockSpec((BLK, 32), lambda i: (i, 0)),
            pl.BlockSpec((BLK, 32), lambda i: (i, 0)),
            pl.BlockSpec((BLK,), lambda i: (i,)),
            pl.BlockSpec((64,), lambda i: (0,)),
            pl.BlockSpec((BLK,), lambda i: (i,)),
            pl.BlockSpec((64, 128), lambda i: (0, 0)),
            pl.BlockSpec((128,), lambda i: (0,)),
        ],
        out_specs=pl.BlockSpec((G, 128), lambda i: (0, 0)),
        out_shape=jax.ShapeDtypeStruct((G, 128), jnp.float32),
        scratch_shapes=[
            pltpu.VMEM((G, 64), jnp.float32),
            pltpu.VMEM((G,), jnp.float32),
        ],
    )(a0, a1, y0, y1, dinv, b2, batch_pad, final_W, final_b)


def kernel(x, edge_index, batch, emb_table, depth_table, proj_W, proj_b,
           g1_W, g1_b, g2_W, g2_b, final_W, final_b):
    x_pad = jnp.pad(x, ((0, N_PAD - N_NODES), (0, 0)))
    batch_pad = jnp.pad(batch, (0, N_PAD - N_NODES), constant_values=G)
    src_flat = jnp.pad(edge_index[0], (0, E_PAD - E), constant_values=N_NODES)
    dst_flat = jnp.pad(edge_index[1], (0, E_PAD - E), constant_values=N_NODES)
    zeros_sl = jnp.zeros((SLICE, 32), jnp.float32)

    deg_parts = _deg_kernel(dst_flat)
    y0, y1, dinv = _embed_call(x_pad, deg_parts, emb_table, depth_table,
                               proj_W, proj_b, g1_W)
    a0, a1 = _conv_kernel(y0, y1, src_flat, dst_flat, zeros_sl)
    z0, z1 = _post1_call(a0, a1, y0, y1, dinv, g1_b, g2_W)
    c0, c1 = _conv_kernel(z0, z1, src_flat, dst_flat, zeros_sl)
    return _post2_call(c0, c1, z0, z1, dinv, g2_b, batch_pad, final_W, final_b)


# speedup vs baseline: 31.7233x; 1.3172x over previous
import functools

import jax
import jax.numpy as jnp
from jax import lax
from jax.experimental import pallas as pl
from jax.experimental.pallas import tpu as pltpu
from jax.experimental.pallas import tpu_sc as plsc

N_NODES = 50000
N_PAD = 51200
E = 800000
ER = 6272
E_PAD = ER * 128
G = 256
NC, NS = 2, 16
EG = 256
E_PS = E_PAD // NS
NG = E_PS // EG
NOUT = (NG + 2 + 2) // 3
SLICE = N_PAD // NS
BLK = 2048
GRID = N_PAD // BLK

def _mesh():
    return plsc.VectorSubcoreMesh(core_axis_name="c", subcore_axis_name="s",
                                  num_cores=NC, num_subcores=NS)


DEG_CHUNK = E_PAD // 32


def _deg_body(dst_hbm, out_hbm, stage, hist):
    c = lax.axis_index("c")
    s = lax.axis_index("s")
    wid = s * NC + c

    zero16 = jnp.zeros((16,), jnp.float32)
    ones16 = jnp.ones((16,), jnp.float32)

    def zb(i, _):
        hist[0, pl.ds(i * 16, 16)] = zero16
        return 0
    lax.fori_loop(0, N_PAD // 16, zb, 0)

    pltpu.sync_copy(dst_hbm.at[pl.ds(wid * DEG_CHUNK, DEG_CHUNK)], stage)

    def kloop(k, _):
        idx = stage[pl.ds(k * 16, 16)]
        plsc.addupdate_scatter(hist.at[0], [idx], ones16)
        return 0
    lax.fori_loop(0, DEG_CHUNK // 16, kloop, 0)

    pltpu.sync_copy(hist, out_hbm.at[wid])


@functools.lru_cache(maxsize=None)
def _deg_kernel_fn():
    return pl.kernel(
        _deg_body,
        mesh=_mesh(),
        out_type=jax.ShapeDtypeStruct((32, 1, N_PAD), jnp.float32),
        scratch_types=[
            pltpu.VMEM((DEG_CHUNK,), jnp.int32),
            pltpu.VMEM((1, N_PAD), jnp.float32),
        ],
        compiler_params=pltpu.CompilerParams(needs_layout_passes=False, use_tc_tiling_on_sc=False),
    )


def _deg_kernel(dst_flat):
    return _deg_kernel_fn()(dst_flat)


def _conv_body(y0_hbm, y1_hbm, src_hbm, dst_hbm, zeros_hbm, a0_hbm, a1_hbm,
               accsh, sg0, sg1, sg2, dg0, dg1, dg2, rb0, rb1, rb2,
               si0, si1, si2, sgm0, sgm1, sgm2, ssm0, ssm1, ssm2):
    c = lax.axis_index("c")
    s = lax.axis_index("s")
    srcg = [sg0, sg1, sg2]
    dstg = [dg0, dg1, dg2]
    rowb = [rb0, rb1, rb2]
    sem_i = [si0, si1, si2]
    sem_g = [sgm0, sgm1, sgm2]
    sem_s = [ssm0, ssm1, ssm2]

    pltpu.sync_copy(zeros_hbm, accsh.at[pl.ds(s * SLICE, SLICE)])
    plsc.subcore_barrier()

    ebase = s * E_PS

    def idx_issue(g, slot):
        e0 = ebase + g * EG
        pltpu.async_copy(src_hbm.at[pl.ds(e0, EG)], srcg[slot], sem_i[slot])
        pltpu.async_copy(dst_hbm.at[pl.ds(e0, EG)], dstg[slot], sem_i[slot])

    def idx_wait(slot):
        pltpu.make_async_copy(src_hbm.at[pl.ds(0, EG)], srcg[slot],
                              sem_i[slot]).wait()
        pltpu.make_async_copy(dst_hbm.at[pl.ds(0, EG)], dstg[slot],
                              sem_i[slot]).wait()

    def run(yc, ac):
        def gat_issue(slot):
            pltpu.async_copy(yc.at[srcg[slot]], rowb[slot], sem_g[slot])

        def gat_wait(slot):
            pltpu.make_async_copy(yc.at[srcg[slot]], rowb[slot],
                                  sem_g[slot]).wait()

        def sct_issue(slot):
            pltpu.async_copy(rowb[slot], accsh.at[dstg[slot]], sem_s[slot],
                             add=True)

        def sct_wait(slot):
            pltpu.make_async_copy(rowb[slot], accsh.at[dstg[slot]],
                                  sem_s[slot]).wait()

        idx_issue(0, 0)
        idx_issue(1, 1)
        idx_wait(0)
        gat_issue(0)

        def outer(j, _):
            for b in range(3):
                g = 3 * j + b
                qs = (b + 1) % 3
                rs = (b + 2) % 3

                @pl.when(g < NG - 1)
                def _():
                    idx_wait(qs)
                    gat_issue(qs)

                @pl.when(g < NG)
                def _():
                    gat_wait(b)
                    sct_issue(b)

                @pl.when(g < NG - 2)
                def _():
                    @pl.when(g >= 1)
                    def _():
                        sct_wait(rs)
                    idx_issue(g + 2, rs)
            return 0
        lax.fori_loop(0, NOUT, outer, 0)

        sct_wait(0)
        sct_wait(1)
        sct_wait(2)
        plsc.subcore_barrier()
        pltpu.sync_copy(accsh.at[pl.ds(s * SLICE, SLICE)],
                        ac.at[pl.ds(s * SLICE, SLICE)])

    @pl.when(c == 0)
    def _():
        run(y0_hbm, a0_hbm)

    @pl.when(c == 1)
    def _():
        run(y1_hbm, a1_hbm)


@functools.lru_cache(maxsize=None)
def _conv_kernel_fn():
    return pl.kernel(
        _conv_body,
        mesh=_mesh(),
        out_type=(jax.ShapeDtypeStruct((N_PAD, 32), jnp.float32),
                  jax.ShapeDtypeStruct((N_PAD, 32), jnp.float32)),
        scratch_types=[
            pltpu.VMEM_SHARED((N_PAD, 32), jnp.float32),
            pltpu.VMEM((EG,), jnp.int32),
            pltpu.VMEM((EG,), jnp.int32),
            pltpu.VMEM((EG,), jnp.int32),
            pltpu.VMEM((EG,), jnp.int32),
            pltpu.VMEM((EG,), jnp.int32),
            pltpu.VMEM((EG,), jnp.int32),
            pltpu.VMEM((EG, 32), jnp.float32),
            pltpu.VMEM((EG, 32), jnp.float32),
            pltpu.VMEM((EG, 32), jnp.float32),
            pltpu.SemaphoreType.DMA,
            pltpu.SemaphoreType.DMA,
            pltpu.SemaphoreType.DMA,
            pltpu.SemaphoreType.DMA,
            pltpu.SemaphoreType.DMA,
            pltpu.SemaphoreType.DMA,
            pltpu.SemaphoreType.DMA,
            pltpu.SemaphoreType.DMA,
            pltpu.SemaphoreType.DMA,
        ],
        compiler_params=pltpu.CompilerParams(needs_layout_passes=False, use_tc_tiling_on_sc=False),
    )


def _conv_kernel(y0, y1, src_flat, dst_flat, zeros_sl):
    return _conv_kernel_fn()(y0, y1, src_flat, dst_flat, zeros_sl)


def _embed_body(x_ref, degp_ref, emb_ref, dep_ref, pw_ref, pb_ref, w1_ref,
                y0_ref, y1_ref, dinv_ref):
    xi = x_ref[...]
    comb0 = xi[:, 0]
    comb1 = jnp.clip(xi[:, 1], 0, 19)
    t1 = jnp.concatenate(
        [emb_ref[...] @ pw_ref[:16, :], jnp.zeros((7, 64), jnp.float32)], axis=0)
    t2 = jnp.concatenate(
        [dep_ref[...] @ pw_ref[16:, :], jnp.zeros((12, 64), jnp.float32)], axis=0)
    oh0 = (lax.broadcasted_iota(jnp.int32, (BLK, 16), 1)
           == comb0[:, None]).astype(jnp.float32)
    oh1 = (lax.broadcasted_iota(jnp.int32, (BLK, 32), 1)
           == comb1[:, None]).astype(jnp.float32)
    h0 = (oh0 @ t1) + (oh1 @ t2) + pb_ref[...][None, :]
    xw1 = h0 @ w1_ref[...]
    deg = jnp.sum(degp_ref[...], axis=(0, 1))
    dinv = lax.rsqrt(deg + 1.0)
    y = xw1 * dinv[:, None]
    y0_ref[...] = y[:, :32]
    y1_ref[...] = y[:, 32:]
    dinv_ref[...] = dinv


def _embed_call(x_pad, deg_parts, emb_table, depth_table, proj_W, proj_b, g1_W):
    return pl.pallas_call(
        _embed_body,
        grid=(GRID,),
        in_specs=[
            pl.BlockSpec((BLK, 2), lambda i: (i, 0)),
            pl.BlockSpec((32, 1, BLK), lambda i: (0, 0, i)),
            pl.BlockSpec((9, 16), lambda i: (0, 0)),
            pl.BlockSpec((20, 16), lambda i: (0, 0)),
            pl.BlockSpec((32, 64), lambda i: (0, 0)),
            pl.BlockSpec((64,), lambda i: (0,)),
            pl.BlockSpec((64, 64), lambda i: (0, 0)),
        ],
        out_specs=[
            pl.BlockSpec((BLK, 32), lambda i: (i, 0)),
            pl.BlockSpec((BLK, 32), lambda i: (i, 0)),
            pl.BlockSpec((BLK,), lambda i: (i,)),
        ],
        out_shape=[
            jax.ShapeDtypeStruct((N_PAD, 32), jnp.float32),
            jax.ShapeDtypeStruct((N_PAD, 32), jnp.float32),
            jax.ShapeDtypeStruct((N_PAD,), jnp.float32),
        ],
    )(x_pad, deg_parts, emb_table, depth_table, proj_W, proj_b, g1_W)


def _post1_body(a0_ref, a1_ref, y0_ref, y1_ref, dinv_ref, b_ref, w_ref,
                z0_ref, z1_ref):
    dinv = dinv_ref[...]
    h = jnp.concatenate([a0_ref[...] + y0_ref[...],
                         a1_ref[...] + y1_ref[...]], axis=1)
    h = jnp.maximum(h * dinv[:, None] + b_ref[...][None, :], 0.0)
    y2 = (h @ w_ref[...]) * dinv[:, None]
    z0_ref[...] = y2[:, :32]
    z1_ref[...] = y2[:, 32:]


def _post1_call(a0, a1, y0, y1, dinv, b1, g2_W):
    return pl.pallas_call(
        _post1_body,
        grid=(GRID,),
        in_specs=[
            pl.BlockSpec((BLK, 32), lambda i: (i, 0)),
            pl.BlockSpec((BLK, 32), lambda i: (i, 0)),
            pl.BlockSpec((BLK, 32), lambda i: (i, 0)),
            pl.BlockSpec((BLK, 32), lambda i: (i, 0)),
            pl.BlockSpec((BLK,), lambda i: (i,)),
            pl.BlockSpec((64,), lambda i: (0,)),
            pl.BlockSpec((64, 64), lambda i: (0, 0)),
        ],
        out_specs=[
            pl.BlockSpec((BLK, 32), lambda i: (i, 0)),
            pl.BlockSpec((BLK, 32), lambda i: (i, 0)),
        ],
        out_shape=[
            jax.ShapeDtypeStruct((N_PAD, 32), jnp.float32),
            jax.ShapeDtypeStruct((N_PAD, 32), jnp.float32),
        ],
    )(a0, a1, y0, y1, dinv, b1, g2_W)


def _post2_body(a0_ref, a1_ref, y0_ref, y1_ref, dinv_ref, b_ref, batch_ref,
                fw_ref, fb_ref, out_ref, sacc, cacc):
    i = pl.program_id(0)

    @pl.when(i == 0)
    def _():
        sacc[...] = jnp.zeros_like(sacc)
        cacc[...] = jnp.zeros_like(cacc)

    dinv = dinv_ref[...]
    h = jnp.concatenate([a0_ref[...] + y0_ref[...],
                         a1_ref[...] + y1_ref[...]], axis=1)
    h = jnp.maximum(h * dinv[:, None] + b_ref[...][None, :], 0.0)
    ohT = (lax.broadcasted_iota(jnp.int32, (G, BLK), 0)
           == batch_ref[...][None, :]).astype(jnp.float32)
    sacc[...] += ohT @ h
    cacc[...] += jnp.sum(ohT, axis=1)
    pooled = sacc[...] / jnp.maximum(cacc[...], 1.0)[:, None]
    out_ref[...] = pooled @ fw_ref[...] + fb_ref[...][None, :]


def _post2_call(a0, a1, y0, y1, dinv, b2, batch_pad, final_W, final_b):
    return pl.pallas_call(
        _post2_body,
        grid=(GRID,),
        in_specs=[
            pl.BlockSpec((BLK, 32), lambda i: (i, 0)),
            pl.BlockSpec((BLK, 32), lambda i: (i, 0)),
            pl.BlockSpec((BLK, 32), lambda i: (i, 0)),
            pl.BlockSpec((BLK, 32), lambda i: (i, 0)),
            pl.BlockSpec((BLK,), lambda i: (i,)),
            pl.BlockSpec((64,), lambda i: (0,)),
            pl.BlockSpec((BLK,), lambda i: (i,)),
            pl.BlockSpec((64, 128), lambda i: (0, 0)),
            pl.BlockSpec((128,), lambda i: (0,)),
        ],
        out_specs=pl.BlockSpec((G, 128), lambda i: (0, 0)),
        out_shape=jax.ShapeDtypeStruct((G, 128), jnp.float32),
        scratch_shapes=[
            pltpu.VMEM((G, 64), jnp.float32),
            pltpu.VMEM((G,), jnp.float32),
        ],
    )(a0, a1, y0, y1, dinv, b2, batch_pad, final_W, final_b)


def kernel(x, edge_index, batch, emb_table, depth_table, proj_W, proj_b,
           g1_W, g1_b, g2_W, g2_b, final_W, final_b):
    x_pad = jnp.pad(x, ((0, N_PAD - N_NODES), (0, 0)))
    batch_pad = jnp.pad(batch, (0, N_PAD - N_NODES), constant_values=G)
    src_flat = jnp.pad(edge_index[0], (0, E_PAD - E), constant_values=N_NODES)
    dst_flat = jnp.pad(edge_index[1], (0, E_PAD - E), constant_values=N_NODES)
    zeros_sl = jnp.zeros((SLICE, 32), jnp.float32)

    deg_parts = _deg_kernel(dst_flat)
    y0, y1, dinv = _embed_call(x_pad, deg_parts, emb_table, depth_table,
                               proj_W, proj_b, g1_W)
    a0, a1 = _conv_kernel(y0, y1, src_flat, dst_flat, zeros_sl)
    z0, z1 = _post1_call(a0, a1, y0, y1, dinv, g1_b, g2_W)
    c0, c1 = _conv_kernel(z0, z1, src_flat, dst_flat, zeros_sl)
    return _post2_call(c0, c1, z0, z1, dinv, g2_b, batch_pad, final_W, final_b)


# speedup vs baseline: 39.7510x; 1.2531x over previous
import functools

import jax
import jax.numpy as jnp
from jax import lax
from jax.experimental import pallas as pl
from jax.experimental.pallas import tpu as pltpu
from jax.experimental.pallas import tpu_sc as plsc

N_NODES = 50000
N_PAD = 51200
E = 800000
G = 256
NC, NS = 2, 16
EG = 256
NGRP = E // EG
NG_BASE = NGRP // NS
NG_REM = NGRP - NS * NG_BASE
NOUT = (NG_BASE + 1 + 2 + 2) // 3
SLICE = N_PAD // NS
BLK = 2048
GRID = N_PAD // BLK

_SC_PARAMS = dict(needs_layout_passes=False, use_tc_tiling_on_sc=False)


def _mesh():
    return plsc.VectorSubcoreMesh(core_axis_name="c", subcore_axis_name="s",
                                  num_cores=NC, num_subcores=NS)


DEG_CHUNK = E // 32
DEG_FULL = DEG_CHUNK // 16


def _deg_body(edge_hbm, out_hbm, stage, hist):
    c = lax.axis_index("c")
    s = lax.axis_index("s")
    wid = s * NC + c

    zero16 = jnp.zeros((16,), jnp.float32)
    ones16 = jnp.ones((16,), jnp.float32)

    def zb(i, _):
        hist[pl.ds(i * 16, 16)] = zero16
        return 0
    lax.fori_loop(0, N_PAD // 16, zb, 0)

    pltpu.sync_copy(edge_hbm.at[1, pl.ds(wid * DEG_CHUNK, DEG_CHUNK)],
                    stage.at[pl.ds(0, DEG_CHUNK)])

    def kloop(k, _):
        idx = stage[pl.ds(k * 16, 16)]
        plsc.addupdate_scatter(hist, [idx], ones16)
        return 0
    lax.fori_loop(0, DEG_FULL, kloop, 0)
    tail_idx = stage[pl.ds(DEG_FULL * 16, 16)]
    tail_mask = lax.iota(jnp.int32, 16) < (DEG_CHUNK - DEG_FULL * 16)
    plsc.addupdate_scatter(hist, [tail_idx], ones16, mask=tail_mask)

    pltpu.sync_copy(hist, out_hbm.at[wid])


@functools.lru_cache(maxsize=None)
def _deg_kernel_fn():
    return pl.kernel(
        _deg_body,
        mesh=_mesh(),
        out_type=jax.ShapeDtypeStruct((32, N_PAD), jnp.float32),
        scratch_types=[
            pltpu.VMEM((DEG_CHUNK + 16,), jnp.int32),
            pltpu.VMEM((N_PAD,), jnp.float32),
        ],
        compiler_params=pltpu.CompilerParams(**_SC_PARAMS),
    )


def _deg_kernel(edge_index):
    return _deg_kernel_fn()(edge_index)


def _conv_body(y2_hbm, edge_hbm, zeros_hbm, acc_hbm,
               accsh, sg0, sg1, sg2, dg0, dg1, dg2, rb0, rb1, rb2,
               si0, si1, si2, sgm0, sgm1, sgm2, ssm0, ssm1, ssm2):
    c = lax.axis_index("c")
    s = lax.axis_index("s")
    srcg = [sg0, sg1, sg2]
    dstg = [dg0, dg1, dg2]
    rowb = [rb0, rb1, rb2]
    sem_i = [si0, si1, si2]
    sem_g = [sgm0, sgm1, sgm2]
    sem_s = [ssm0, ssm1, ssm2]

    pltpu.sync_copy(zeros_hbm, accsh.at[pl.ds(s * SLICE, SLICE)])
    plsc.subcore_barrier()

    ng = NG_BASE + jnp.where(s < NG_REM, 1, 0)
    cvec = jnp.full((16,), c, jnp.int32)

    def idx_issue(j, slot):
        e0 = (s + NS * j) * EG
        pltpu.async_copy(edge_hbm.at[0, pl.ds(e0, EG)], srcg[slot],
                         sem_i[slot])
        pltpu.async_copy(edge_hbm.at[1, pl.ds(e0, EG)], dstg[slot],
                         sem_i[slot])

    def idx_wait(slot):
        pltpu.make_async_copy(edge_hbm.at[0, pl.ds(0, EG)], srcg[slot],
                              sem_i[slot]).wait()
        pltpu.make_async_copy(edge_hbm.at[1, pl.ds(0, EG)], dstg[slot],
                              sem_i[slot]).wait()

    def transform(slot):
        def t(k, _):
            v = srcg[slot][pl.ds(k * 16, 16)]
            srcg[slot][pl.ds(k * 16, 16)] = v + v + cvec
            return 0
        lax.fori_loop(0, EG // 16, t, 0)

    def gat_issue(slot):
        pltpu.async_copy(y2_hbm.at[srcg[slot]], rowb[slot], sem_g[slot])

    def gat_wait(slot):
        pltpu.make_async_copy(y2_hbm.at[srcg[slot]], rowb[slot],
                              sem_g[slot]).wait()

    def sct_issue(slot):
        pltpu.async_copy(rowb[slot], accsh.at[dstg[slot]], sem_s[slot],
                         add=True)

    def sct_wait(slot):
        pltpu.make_async_copy(rowb[slot], accsh.at[dstg[slot]],
                              sem_s[slot]).wait()

    idx_issue(0, 0)
    idx_issue(1, 1)
    idx_wait(0)
    transform(0)
    gat_issue(0)

    def outer(jo, _):
        for b in range(3):
            g = 3 * jo + b
            qs = (b + 1) % 3
            rs = (b + 2) % 3

            @pl.when(g < ng - 1)
            def _():
                idx_wait(qs)
                transform(qs)
                gat_issue(qs)

            @pl.when(g < ng)
            def _():
                gat_wait(b)
                sct_issue(b)

            @pl.when(g < ng - 2)
            def _():
                @pl.when(g >= 1)
                def _():
                    sct_wait(rs)
                idx_issue(g + 2, rs)
        return 0
    lax.fori_loop(0, NOUT, outer, 0)

    sct_wait(0)
    sct_wait(1)
    sct_wait(2)
    plsc.subcore_barrier()
    pltpu.sync_copy(accsh.at[pl.ds(s * SLICE, SLICE)],
                    acc_hbm.at[pl.ds(s * SLICE, SLICE), pl.ds(c * 32, 32)])


@functools.lru_cache(maxsize=None)
def _conv_kernel_fn():
    return pl.kernel(
        _conv_body,
        mesh=_mesh(),
        out_type=jax.ShapeDtypeStruct((N_PAD, 64), jnp.float32),
        scratch_types=[
            pltpu.VMEM_SHARED((N_PAD, 32), jnp.float32),
            pltpu.VMEM((EG,), jnp.int32),
            pltpu.VMEM((EG,), jnp.int32),
            pltpu.VMEM((EG,), jnp.int32),
            pltpu.VMEM((EG,), jnp.int32),
            pltpu.VMEM((EG,), jnp.int32),
            pltpu.VMEM((EG,), jnp.int32),
            pltpu.VMEM((EG, 32), jnp.float32),
            pltpu.VMEM((EG, 32), jnp.float32),
            pltpu.VMEM((EG, 32), jnp.float32),
            pltpu.SemaphoreType.DMA,
            pltpu.SemaphoreType.DMA,
            pltpu.SemaphoreType.DMA,
            pltpu.SemaphoreType.DMA,
            pltpu.SemaphoreType.DMA,
            pltpu.SemaphoreType.DMA,
            pltpu.SemaphoreType.DMA,
            pltpu.SemaphoreType.DMA,
            pltpu.SemaphoreType.DMA,
        ],
        compiler_params=pltpu.CompilerParams(**_SC_PARAMS),
    )


def _conv_kernel(y2, edge_index, zeros_sl):
    return _conv_kernel_fn()(y2, edge_index, zeros_sl)


def _embed_body(x_ref, degp_ref, emb_ref, dep_ref, pw_ref, pb_ref, w1_ref,
                y_ref, dinv_ref):
    xi = x_ref[...]
    comb0 = xi[:, 0]
    comb1 = jnp.clip(xi[:, 1], 0, 19)
    t1 = jnp.concatenate(
        [emb_ref[...] @ pw_ref[:16, :], jnp.zeros((7, 64), jnp.float32)], axis=0)
    t2 = jnp.concatenate(
        [dep_ref[...] @ pw_ref[16:, :], jnp.zeros((12, 64), jnp.float32)], axis=0)
    oh0 = (lax.broadcasted_iota(jnp.int32, (BLK, 16), 1)
           == comb0[:, None]).astype(jnp.float32)
    oh1 = (lax.broadcasted_iota(jnp.int32, (BLK, 32), 1)
           == comb1[:, None]).astype(jnp.float32)
    h0 = (oh0 @ t1) + (oh1 @ t2) + pb_ref[...][None, :]
    xw1 = h0 @ w1_ref[...]
    deg = jnp.sum(degp_ref[...], axis=0)
    dinv = lax.rsqrt(deg + 1.0)
    y_ref[...] = xw1 * dinv[:, None]
    dinv_ref[...] = dinv


def _embed_call(x_pad, deg2, emb_table, depth_table, proj_W, proj_b, g1_W):
    return pl.pallas_call(
        _embed_body,
        grid=(GRID,),
        in_specs=[
            pl.BlockSpec((BLK, 2), lambda i: (i, 0)),
            pl.BlockSpec((32, BLK), lambda i: (0, i)),
            pl.BlockSpec((9, 16), lambda i: (0, 0)),
            pl.BlockSpec((20, 16), lambda i: (0, 0)),
            pl.BlockSpec((32, 64), lambda i: (0, 0)),
            pl.BlockSpec((64,), lambda i: (0,)),
            pl.BlockSpec((64, 64), lambda i: (0, 0)),
        ],
        out_specs=[
            pl.BlockSpec((BLK, 64), lambda i: (i, 0)),
            pl.BlockSpec((BLK,), lambda i: (i,)),
        ],
        out_shape=[
            jax.ShapeDtypeStruct((N_PAD, 64), jnp.float32),
            jax.ShapeDtypeStruct((N_PAD,), jnp.float32),
        ],
    )(x_pad, deg2, emb_table, depth_table, proj_W, proj_b, g1_W)


def _post1_body(a_ref, y_ref, dinv_ref, b_ref, w_ref, z_ref):
    dinv = dinv_ref[...]
    h = (a_ref[...] + y_ref[...]) * dinv[:, None] + b_ref[...][None, :]
    h = jnp.maximum(h, 0.0)
    z_ref[...] = (h @ w_ref[...]) * dinv[:, None]


def _post1_call(a, y, dinv, b1, g2_W):
    return pl.pallas_call(
        _post1_body,
        grid=(GRID,),
        in_specs=[
            pl.BlockSpec((BLK, 64), lambda i: (i, 0)),
            pl.BlockSpec((BLK, 64), lambda i: (i, 0)),
            pl.BlockSpec((BLK,), lambda i: (i,)),
            pl.BlockSpec((64,), lambda i: (0,)),
            pl.BlockSpec((64, 64), lambda i: (0, 0)),
        ],
        out_specs=pl.BlockSpec((BLK, 64), lambda i: (i, 0)),
        out_shape=jax.ShapeDtypeStruct((N_PAD, 64), jnp.float32),
    )(a, y, dinv, b1, g2_W)


def _post2_body(a_ref, y_ref, dinv_ref, b_ref, batch_ref, fw_ref, fb_ref,
                out_ref, sacc, cacc):
    i = pl.program_id(0)

    @pl.when(i == 0)
    def _():
        sacc[...] = jnp.zeros_like(sacc)
        cacc[...] = jnp.zeros_like(cacc)

    dinv = dinv_ref[...]
    h = (a_ref[...] + y_ref[...]) * dinv[:, None] + b_ref[...][None, :]
    h = jnp.maximum(h, 0.0)
    ohT = (lax.broadcasted_iota(jnp.int32, (G, BLK), 0)
           == batch_ref[...][None, :]).astype(jnp.float32)
    sacc[...] += ohT @ h
    cacc[...] += jnp.sum(ohT, axis=1)
    pooled = sacc[...] / jnp.maximum(cacc[...], 1.0)[:, None]
    out_ref[...] = pooled @ fw_ref[...] + fb_ref[...][None, :]


def _post2_call(a, y, dinv, b2, batch_pad, final_W, final_b):
    return pl.pallas_call(
        _post2_body,
        grid=(GRID,),
        in_specs=[
            pl.BlockSpec((BLK, 64), lambda i: (i, 0)),
            pl.BlockSpec((BLK, 64), lambda i: (i, 0)),
            pl.BlockSpec((BLK,), lambda i: (i,)),
            pl.BlockSpec((64,), lambda i: (0,)),
            pl.BlockSpec((BLK,), lambda i: (i,)),
            pl.BlockSpec((64, 128), lambda i: (0, 0)),
            pl.BlockSpec((128,), lambda i: (0,)),
        ],
        out_specs=pl.BlockSpec((G, 128), lambda i: (0, 0)),
        out_shape=jax.ShapeDtypeStruct((G, 128), jnp.float32),
        scratch_shapes=[
            pltpu.VMEM((G, 64), jnp.float32),
            pltpu.VMEM((G,), jnp.float32),
        ],
    )(a, y, dinv, b2, batch_pad, final_W, final_b)


def kernel(x, edge_index, batch, emb_table, depth_table, proj_W, proj_b,
           g1_W, g1_b, g2_W, g2_b, final_W, final_b):
    x_pad = jnp.pad(x, ((0, N_PAD - N_NODES), (0, 0)))
    batch_pad = jnp.pad(batch, (0, N_PAD - N_NODES), constant_values=G)
    zeros_sl = jnp.zeros((SLICE, 32), jnp.float32)

    deg2 = _deg_kernel(edge_index)
    y, dinv = _embed_call(x_pad, deg2, emb_table, depth_table,
                          proj_W, proj_b, g1_W)
    a = _conv_kernel(y.reshape(2 * N_PAD, 32), edge_index, zeros_sl)
    z = _post1_call(a, y, dinv, g1_b, g2_W)
    a2 = _conv_kernel(z.reshape(2 * N_PAD, 32), edge_index, zeros_sl)
    return _post2_call(a2, z, dinv, g2_b, batch_pad, final_W, final_b)


# speedup vs baseline: 44.5800x; 1.1215x over previous
import functools

import jax
import jax.numpy as jnp
from jax import lax
from jax.experimental import pallas as pl
from jax.experimental.pallas import tpu as pltpu
from jax.experimental.pallas import tpu_sc as plsc

N_NODES = 50000
N_PAD = 51200
E = 800000
G = 256
NC, NS = 2, 16
EG = 256
NGRP = E // EG
NG_BASE = NGRP // NS
NG_REM = NGRP - NS * NG_BASE
NOUT = (NG_BASE + 1 + 2 + 2) // 3
SLICE = N_PAD // NS
BLK = 2048
GRID = N_PAD // BLK
M = N_PAD // 2
MBLK = BLK // 2

_SC_PARAMS = dict(needs_layout_passes=False, use_tc_tiling_on_sc=False)


def _mesh():
    return plsc.VectorSubcoreMesh(core_axis_name="c", subcore_axis_name="s",
                                  num_cores=NC, num_subcores=NS)


DEG_CHUNK = E // 32
DEG_FULL = DEG_CHUNK // 16


def _deg_body(edge_hbm, out_hbm, stage, hist):
    c = lax.axis_index("c")
    s = lax.axis_index("s")
    wid = s * NC + c

    zero16 = jnp.zeros((16,), jnp.float32)
    ones16 = jnp.ones((16,), jnp.float32)

    def zb(i, _):
        hist[pl.ds(i * 16, 16)] = zero16
        return 0
    lax.fori_loop(0, N_PAD // 16, zb, 0)

    pltpu.sync_copy(edge_hbm.at[1, pl.ds(wid * DEG_CHUNK, DEG_CHUNK)],
                    stage.at[pl.ds(0, DEG_CHUNK)])

    def pairpos(idx):
        return lax.shift_right_logical(idx, 1) + (idx & 1) * M

    def kloop(k, _):
        idx = stage[pl.ds(k * 16, 16)]
        plsc.addupdate_scatter(hist, [pairpos(idx)], ones16)
        return 0
    lax.fori_loop(0, DEG_FULL, kloop, 0)
    tail_idx = stage[pl.ds(DEG_FULL * 16, 16)]
    tail_mask = lax.iota(jnp.int32, 16) < (DEG_CHUNK - DEG_FULL * 16)
    plsc.addupdate_scatter(hist, [pairpos(tail_idx)], ones16, mask=tail_mask)

    pltpu.sync_copy(hist, out_hbm.at[wid])


@functools.lru_cache(maxsize=None)
def _deg_kernel_fn():
    return pl.kernel(
        _deg_body,
        mesh=_mesh(),
        out_type=jax.ShapeDtypeStruct((32, N_PAD), jnp.float32),
        scratch_types=[
            pltpu.VMEM((DEG_CHUNK + 16,), jnp.int32),
            pltpu.VMEM((N_PAD,), jnp.float32),
        ],
        compiler_params=pltpu.CompilerParams(**_SC_PARAMS),
    )


def _deg_kernel(edge_index):
    return _deg_kernel_fn()(edge_index)


def _conv_body(y2_hbm, edge_hbm, zeros_hbm, acc_hbm,
               accsh, sg0, sg1, sg2, dg0, dg1, dg2, rb0, rb1, rb2,
               si0, si1, si2, sgm0, sgm1, sgm2, ssm0, ssm1, ssm2):
    c = lax.axis_index("c")
    s = lax.axis_index("s")
    srcg = [sg0, sg1, sg2]
    dstg = [dg0, dg1, dg2]
    rowb = [rb0, rb1, rb2]
    sem_i = [si0, si1, si2]
    sem_g = [sgm0, sgm1, sgm2]
    sem_s = [ssm0, ssm1, ssm2]

    pltpu.sync_copy(zeros_hbm, accsh.at[pl.ds(s * SLICE, SLICE)])
    plsc.subcore_barrier()

    ng = NG_BASE + jnp.where(s < NG_REM, 1, 0)
    cvec = jnp.full((16,), c, jnp.int32)

    def idx_issue(j, slot):
        e0 = (s + NS * j) * EG
        pltpu.async_copy(edge_hbm.at[0, pl.ds(e0, EG)], srcg[slot],
                         sem_i[slot])
        pltpu.async_copy(edge_hbm.at[1, pl.ds(e0, EG)], dstg[slot],
                         sem_i[slot])

    def idx_wait(slot):
        pltpu.make_async_copy(edge_hbm.at[0, pl.ds(0, EG)], srcg[slot],
                              sem_i[slot]).wait()
        pltpu.make_async_copy(edge_hbm.at[1, pl.ds(0, EG)], dstg[slot],
                              sem_i[slot]).wait()

    def transform(slot):
        def t(k, _):
            v = srcg[slot][pl.ds(k * 16, 16)]
            srcg[slot][pl.ds(k * 16, 16)] = v + v + cvec
            return 0
        lax.fori_loop(0, EG // 16, t, 0)

    def gat_issue(slot):
        pltpu.async_copy(y2_hbm.at[srcg[slot]], rowb[slot], sem_g[slot])

    def gat_wait(slot):
        pltpu.make_async_copy(y2_hbm.at[srcg[slot]], rowb[slot],
                              sem_g[slot]).wait()

    def sct_issue(slot):
        pltpu.async_copy(rowb[slot], accsh.at[dstg[slot]], sem_s[slot],
                         add=True)

    def sct_wait(slot):
        pltpu.make_async_copy(rowb[slot], accsh.at[dstg[slot]],
                              sem_s[slot]).wait()

    idx_issue(0, 0)
    idx_issue(1, 1)
    idx_wait(0)
    transform(0)
    gat_issue(0)

    def outer(jo, _):
        for b in range(3):
            g = 3 * jo + b
            qs = (b + 1) % 3
            rs = (b + 2) % 3

            @pl.when(g < ng - 1)
            def _():
                idx_wait(qs)
                transform(qs)
                gat_issue(qs)

            @pl.when(g < ng)
            def _():
                gat_wait(b)
                sct_issue(b)

            @pl.when(g < ng - 2)
            def _():
                @pl.when(g >= 1)
                def _():
                    sct_wait(rs)
                idx_issue(g + 2, rs)
        return 0
    lax.fori_loop(0, NOUT, outer, 0)

    sct_wait(0)
    sct_wait(1)
    sct_wait(2)
    plsc.subcore_barrier()
    pltpu.sync_copy(accsh.at[pl.ds(s * SLICE, SLICE)],
                    acc_hbm.at[pl.ds(s * SLICE, SLICE), pl.ds(c * 32, 32)])


@functools.lru_cache(maxsize=None)
def _conv_kernel_fn():
    return pl.kernel(
        _conv_body,
        mesh=_mesh(),
        out_type=jax.ShapeDtypeStruct((N_PAD, 64), jnp.float32),
        scratch_types=[
            pltpu.VMEM_SHARED((N_PAD, 32), jnp.float32),
            pltpu.VMEM((EG,), jnp.int32),
            pltpu.VMEM((EG,), jnp.int32),
            pltpu.VMEM((EG,), jnp.int32),
            pltpu.VMEM((EG,), jnp.int32),
            pltpu.VMEM((EG,), jnp.int32),
            pltpu.VMEM((EG,), jnp.int32),
            pltpu.VMEM((EG, 32), jnp.float32),
            pltpu.VMEM((EG, 32), jnp.float32),
            pltpu.VMEM((EG, 32), jnp.float32),
            pltpu.SemaphoreType.DMA,
            pltpu.SemaphoreType.DMA,
            pltpu.SemaphoreType.DMA,
            pltpu.SemaphoreType.DMA,
            pltpu.SemaphoreType.DMA,
            pltpu.SemaphoreType.DMA,
            pltpu.SemaphoreType.DMA,
            pltpu.SemaphoreType.DMA,
            pltpu.SemaphoreType.DMA,
        ],
        compiler_params=pltpu.CompilerParams(**_SC_PARAMS),
    )


def _conv_kernel(y2, edge_index, zeros_sl):
    return _conv_kernel_fn()(y2, edge_index, zeros_sl)


def _embed_body(xq_ref, dege_ref, dego_ref, emb_ref, dep_ref, pw_ref, pb_ref,
                w1_ref, y_ref, dinv_ref):
    t1 = jnp.concatenate(
        [emb_ref[...] @ pw_ref[:16, :], jnp.zeros((7, 64), jnp.float32)], axis=0)
    t2 = jnp.concatenate(
        [dep_ref[...] @ pw_ref[16:, :], jnp.zeros((12, 64), jnp.float32)], axis=0)
    xq = xq_ref[...]
    w1 = w1_ref[...]

    def node_y(comb0, comb1, deg):
        oh0 = (lax.broadcasted_iota(jnp.int32, (MBLK, 16), 1)
               == comb0[:, None]).astype(jnp.float32)
        oh1 = (lax.broadcasted_iota(jnp.int32, (MBLK, 32), 1)
               == jnp.clip(comb1, 0, 19)[:, None]).astype(jnp.float32)
        h0 = (oh0 @ t1) + (oh1 @ t2) + pb_ref[...][None, :]
        dinv = lax.rsqrt(deg + 1.0)
        return (h0 @ w1) * dinv[:, None], dinv

    ye, de = node_y(xq[0], xq[2], jnp.sum(dege_ref[...], axis=0))
    yo, do = node_y(xq[1], xq[3], jnp.sum(dego_ref[...], axis=0))
    y_ref[...] = jnp.concatenate([ye, yo], axis=1)
    dinv_ref[...] = jnp.stack([de, do])


def _embed_call(xq, deg_parts, emb_table, depth_table, proj_W, proj_b, g1_W):
    return pl.pallas_call(
        _embed_body,
        grid=(GRID,),
        in_specs=[
            pl.BlockSpec((4, MBLK), lambda i: (0, i)),
            pl.BlockSpec((32, MBLK), lambda i: (0, i)),
            pl.BlockSpec((32, MBLK), lambda i: (0, i + GRID)),
            pl.BlockSpec((9, 16), lambda i: (0, 0)),
            pl.BlockSpec((20, 16), lambda i: (0, 0)),
            pl.BlockSpec((32, 64), lambda i: (0, 0)),
            pl.BlockSpec((64,), lambda i: (0,)),
            pl.BlockSpec((64, 64), lambda i: (0, 0)),
        ],
        out_specs=[
            pl.BlockSpec((MBLK, 128), lambda i: (i, 0)),
            pl.BlockSpec((2, MBLK), lambda i: (0, i)),
        ],
        out_shape=[
            jax.ShapeDtypeStruct((M, 128), jnp.float32),
            jax.ShapeDtypeStruct((2, M), jnp.float32),
        ],
    )(xq, deg_parts, deg_parts, emb_table, depth_table, proj_W, proj_b, g1_W)


def _pair_scale(dinv_ref):
    dv = dinv_ref[...]
    de = jnp.broadcast_to(dv[0][:, None], (MBLK, 64))
    do = jnp.broadcast_to(dv[1][:, None], (MBLK, 64))
    return jnp.concatenate([de, do], axis=1)


def _blockdiag(w):
    z = jnp.zeros((64, 64), jnp.float32)
    return jnp.concatenate([jnp.concatenate([w, z], axis=1),
                            jnp.concatenate([z, w], axis=1)], axis=0)


def _post1_body(a_ref, y_ref, dinv_ref, b_ref, w_ref, z_ref):
    scale = _pair_scale(dinv_ref)
    b2 = jnp.concatenate([b_ref[...], b_ref[...]])
    h = (a_ref[...] + y_ref[...]) * scale + b2[None, :]
    h = jnp.maximum(h, 0.0)
    z_ref[...] = (h @ _blockdiag(w_ref[...])) * scale


def _post1_call(a, y, dinvT, b1, g2_W):
    return pl.pallas_call(
        _post1_body,
        grid=(GRID,),
        in_specs=[
            pl.BlockSpec((MBLK, 128), lambda i: (i, 0)),
            pl.BlockSpec((MBLK, 128), lambda i: (i, 0)),
            pl.BlockSpec((2, MBLK), lambda i: (0, i)),
            pl.BlockSpec((64,), lambda i: (0,)),
            pl.BlockSpec((64, 64), lambda i: (0, 0)),
        ],
        out_specs=pl.BlockSpec((MBLK, 128), lambda i: (i, 0)),
        out_shape=jax.ShapeDtypeStruct((M, 128), jnp.float32),
    )(a, y, dinvT, b1, g2_W)


def _post2_body(a_ref, y_ref, dinv_ref, b_ref, batch_ref, fw_ref, fb_ref,
                out_ref, sacc, cacc):
    i = pl.program_id(0)

    @pl.when(i == 0)
    def _():
        sacc[...] = jnp.zeros_like(sacc)
        cacc[...] = jnp.zeros_like(cacc)

    scale = _pair_scale(dinv_ref)
    b2 = jnp.concatenate([b_ref[...], b_ref[...]])
    h = (a_ref[...] + y_ref[...]) * scale + b2[None, :]
    h = jnp.maximum(h, 0.0)
    bt = batch_ref[...]
    ohTe = (lax.broadcasted_iota(jnp.int32, (G, MBLK), 0)
            == bt[0][None, :]).astype(jnp.float32)
    ohTo = (lax.broadcasted_iota(jnp.int32, (G, MBLK), 0)
            == bt[1][None, :]).astype(jnp.float32)
    sacc[...] += ohTe @ h[:, :64] + ohTo @ h[:, 64:]
    cacc[...] += jnp.sum(ohTe, axis=1) + jnp.sum(ohTo, axis=1)
    pooled = sacc[...] / jnp.maximum(cacc[...], 1.0)[:, None]
    out_ref[...] = pooled @ fw_ref[...] + fb_ref[...][None, :]


def _post2_call(a, y, dinvT, b2, batchT, final_W, final_b):
    return pl.pallas_call(
        _post2_body,
        grid=(GRID,),
        in_specs=[
            pl.BlockSpec((MBLK, 128), lambda i: (i, 0)),
            pl.BlockSpec((MBLK, 128), lambda i: (i, 0)),
            pl.BlockSpec((2, MBLK), lambda i: (0, i)),
            pl.BlockSpec((64,), lambda i: (0,)),
            pl.BlockSpec((2, MBLK), lambda i: (0, i)),
            pl.BlockSpec((64, 128), lambda i: (0, 0)),
            pl.BlockSpec((128,), lambda i: (0,)),
        ],
        out_specs=pl.BlockSpec((G, 128), lambda i: (0, 0)),
        out_shape=jax.ShapeDtypeStruct((G, 128), jnp.float32),
        scratch_shapes=[
            pltpu.VMEM((G, 64), jnp.float32),
            pltpu.VMEM((G,), jnp.float32),
        ],
    )(a, y, dinvT, b2, batchT, final_W, final_b)


def kernel(x, edge_index, batch, emb_table, depth_table, proj_W, proj_b,
           g1_W, g1_b, g2_W, g2_b, final_W, final_b):
    xT = jnp.pad(x.T, ((0, 0), (0, N_PAD - N_NODES)))
    xq = xT.reshape(2, M, 2).transpose(0, 2, 1).reshape(4, M)
    batchT = jnp.pad(batch, (0, N_PAD - N_NODES),
                     constant_values=G).reshape(M, 2).T
    zeros_sl = jnp.zeros((SLICE, 32), jnp.float32)

    deg_parts = _deg_kernel(edge_index)
    y, dinvT = _embed_call(xq, deg_parts, emb_table, depth_table,
                           proj_W, proj_b, g1_W)
    a = _conv_kernel(y.reshape(2 * N_PAD, 32), edge_index, zeros_sl)
    z = _post1_call(a.reshape(M, 128), y, dinvT, g1_b, g2_W)
    a2 = _conv_kernel(z.reshape(2 * N_PAD, 32), edge_index, zeros_sl)
    return _post2_call(a2.reshape(M, 128), z, dinvT, g2_b, batchT,
                       final_W, final_b)


# speedup vs baseline: 45.2398x; 1.0148x over previous
import functools

import jax
import jax.numpy as jnp
from jax import lax
from jax.experimental import pallas as pl
from jax.experimental.pallas import tpu as pltpu
from jax.experimental.pallas import tpu_sc as plsc

N_NODES = 50000
N_PAD = 51200
E = 800000
G = 256
NC, NS = 2, 16
EG = 256
NGRP = E // EG
NG_BASE = NGRP // NS
NG_REM = NGRP - NS * NG_BASE
NOUT = (NG_BASE + 1 + 2 + 2) // 3
SLICE = N_PAD // NS
BLK = 2048
GRID = N_PAD // BLK
M = N_PAD // 2
MBLK = BLK // 2

_SC_PARAMS = dict(needs_layout_passes=False, use_tc_tiling_on_sc=False)


def _mesh():
    return plsc.VectorSubcoreMesh(core_axis_name="c", subcore_axis_name="s",
                                  num_cores=NC, num_subcores=NS)


DEG_CHUNK = E // 32
DEG_FULL = DEG_CHUNK // 16


DROWS = N_PAD // 128
DR_PS = DROWS // NS


def _deg_body(edge_hbm, out_hbm, stage, hist, iota_r, degacc):
    c = lax.axis_index("c")
    s = lax.axis_index("s")
    wid = s * NC + c

    zero16 = jnp.zeros((16,), jnp.float32)
    ones16 = jnp.ones((16,), jnp.float32)

    def zb(i, _):
        hist[lax.div(i, jnp.int32(8)), pl.ds(lax.rem(i, jnp.int32(8)) * 16, 16)] = zero16
        return 0
    lax.fori_loop(0, DROWS * 8, zb, 0)

    def ib(i, _):
        iota_r[pl.ds(i * 16, 16)] = lax.iota(jnp.int32, 16) + i * 16
        return 0
    lax.fori_loop(0, DROWS // 16, ib, 0)

    pltpu.sync_copy(hist.at[pl.ds(0, DR_PS)], degacc.at[pl.ds(s * DR_PS, DR_PS)])
    plsc.subcore_barrier()

    pltpu.sync_copy(edge_hbm.at[1, pl.ds(wid * DEG_CHUNK, DEG_CHUNK)],
                    stage.at[pl.ds(0, DEG_CHUNK)])

    def scatter(idx, mask=None):
        pp = lax.shift_right_logical(idx, 1) + (idx & 1) * M
        plsc.addupdate_scatter(
            hist, [lax.shift_right_logical(pp, 7), pp & 127], ones16,
            mask=mask)

    def kloop(k, _):
        scatter(stage[pl.ds(k * 16, 16)])
        return 0
    lax.fori_loop(0, DEG_FULL, kloop, 0)
    scatter(stage[pl.ds(DEG_FULL * 16, 16)],
            lax.iota(jnp.int32, 16) < (DEG_CHUNK - DEG_FULL * 16))

    plsc.subcore_barrier()
    pltpu.sync_copy(hist, degacc.at[iota_r], add=True)
    plsc.subcore_barrier()
    pltpu.sync_copy(degacc.at[pl.ds(s * DR_PS, DR_PS)],
                    out_hbm.at[c, pl.ds(s * DR_PS, DR_PS)])


@functools.lru_cache(maxsize=None)
def _deg_kernel_fn():
    return pl.kernel(
        _deg_body,
        mesh=_mesh(),
        out_type=jax.ShapeDtypeStruct((NC, DROWS, 128), jnp.float32),
        scratch_types=[
            pltpu.VMEM((DEG_CHUNK + 16,), jnp.int32),
            pltpu.VMEM((DROWS, 128), jnp.float32),
            pltpu.VMEM((DROWS,), jnp.int32),
            pltpu.VMEM_SHARED((DROWS, 128), jnp.float32),
        ],
        compiler_params=pltpu.CompilerParams(**_SC_PARAMS),
    )


def _deg_kernel(edge_index):
    return _deg_kernel_fn()(edge_index)


def _conv_body(y2_hbm, edge_hbm, zeros_hbm, acc_hbm,
               accsh, sg0, sg1, sg2, dg0, dg1, dg2, rb0, rb1, rb2,
               si0, si1, si2, sgm0, sgm1, sgm2, ssm0, ssm1, ssm2):
    c = lax.axis_index("c")
    s = lax.axis_index("s")
    srcg = [sg0, sg1, sg2]
    dstg = [dg0, dg1, dg2]
    rowb = [rb0, rb1, rb2]
    sem_i = [si0, si1, si2]
    sem_g = [sgm0, sgm1, sgm2]
    sem_s = [ssm0, ssm1, ssm2]

    pltpu.sync_copy(zeros_hbm, accsh.at[pl.ds(s * SLICE, SLICE)])
    plsc.subcore_barrier()

    ng = NG_BASE + jnp.where(s < NG_REM, 1, 0)
    cvec = jnp.full((16,), c, jnp.int32)

    def idx_issue(j, slot):
        e0 = (s + NS * j) * EG
        pltpu.async_copy(edge_hbm.at[0, pl.ds(e0, EG)], srcg[slot],
                         sem_i[slot])
        pltpu.async_copy(edge_hbm.at[1, pl.ds(e0, EG)], dstg[slot],
                         sem_i[slot])

    def idx_wait(slot):
        pltpu.make_async_copy(edge_hbm.at[0, pl.ds(0, EG)], srcg[slot],
                              sem_i[slot]).wait()
        pltpu.make_async_copy(edge_hbm.at[1, pl.ds(0, EG)], dstg[slot],
                              sem_i[slot]).wait()

    def transform(slot):
        def t(k, _):
            v = srcg[slot][pl.ds(k * 16, 16)]
            srcg[slot][pl.ds(k * 16, 16)] = v + v + cvec
            return 0
        lax.fori_loop(0, EG // 16, t, 0)

    def gat_issue(slot):
        pltpu.async_copy(y2_hbm.at[srcg[slot]], rowb[slot], sem_g[slot])

    def gat_wait(slot):
        pltpu.make_async_copy(y2_hbm.at[srcg[slot]], rowb[slot],
                              sem_g[slot]).wait()

    def sct_issue(slot):
        pltpu.async_copy(rowb[slot], accsh.at[dstg[slot]], sem_s[slot],
                         add=True)

    def sct_wait(slot):
        pltpu.make_async_copy(rowb[slot], accsh.at[dstg[slot]],
                              sem_s[slot]).wait()

    idx_issue(0, 0)
    idx_issue(1, 1)
    idx_wait(0)
    transform(0)
    gat_issue(0)

    def outer(jo, _):
        for b in range(3):
            g = 3 * jo + b
            qs = (b + 1) % 3
            rs = (b + 2) % 3

            @pl.when(g < ng - 1)
            def _():
                idx_wait(qs)
                transform(qs)
                gat_issue(qs)

            @pl.when(g < ng)
            def _():
                gat_wait(b)
                sct_issue(b)

            @pl.when(g < ng - 2)
            def _():
                @pl.when(g >= 1)
                def _():
                    sct_wait(rs)
                idx_issue(g + 2, rs)
        return 0
    lax.fori_loop(0, NOUT, outer, 0)

    sct_wait(0)
    sct_wait(1)
    sct_wait(2)
    plsc.subcore_barrier()
    pltpu.sync_copy(accsh.at[pl.ds(s * SLICE, SLICE)],
                    acc_hbm.at[pl.ds(s * SLICE, SLICE), pl.ds(c * 32, 32)])


@functools.lru_cache(maxsize=None)
def _conv_kernel_fn():
    return pl.kernel(
        _conv_body,
        mesh=_mesh(),
        out_type=jax.ShapeDtypeStruct((N_PAD, 64), jnp.float32),
        scratch_types=[
            pltpu.VMEM_SHARED((N_PAD, 32), jnp.float32),
            pltpu.VMEM((EG,), jnp.int32),
            pltpu.VMEM((EG,), jnp.int32),
            pltpu.VMEM((EG,), jnp.int32),
            pltpu.VMEM((EG,), jnp.int32),
            pltpu.VMEM((EG,), jnp.int32),
            pltpu.VMEM((EG,), jnp.int32),
            pltpu.VMEM((EG, 32), jnp.float32),
            pltpu.VMEM((EG, 32), jnp.float32),
            pltpu.VMEM((EG, 32), jnp.float32),
            pltpu.SemaphoreType.DMA,
            pltpu.SemaphoreType.DMA,
            pltpu.SemaphoreType.DMA,
            pltpu.SemaphoreType.DMA,
            pltpu.SemaphoreType.DMA,
            pltpu.SemaphoreType.DMA,
            pltpu.SemaphoreType.DMA,
            pltpu.SemaphoreType.DMA,
            pltpu.SemaphoreType.DMA,
        ],
        compiler_params=pltpu.CompilerParams(**_SC_PARAMS),
    )


def _conv_kernel(y2, edge_index, zeros_sl):
    return _conv_kernel_fn()(y2, edge_index, zeros_sl)


def _embed_body(xq_ref, dege_ref, dego_ref, emb_ref, dep_ref, pw_ref, pb_ref,
                w1_ref, y_ref, dinv_ref):
    t1 = jnp.concatenate(
        [emb_ref[...] @ pw_ref[:16, :], jnp.zeros((7, 64), jnp.float32)], axis=0)
    t2 = jnp.concatenate(
        [dep_ref[...] @ pw_ref[16:, :], jnp.zeros((12, 64), jnp.float32)], axis=0)
    xq = xq_ref[...]
    w1 = w1_ref[...]

    def node_y(comb0, comb1, deg):
        oh0 = (lax.broadcasted_iota(jnp.int32, (MBLK, 16), 1)
               == comb0[:, None]).astype(jnp.float32)
        oh1 = (lax.broadcasted_iota(jnp.int32, (MBLK, 32), 1)
               == jnp.clip(comb1, 0, 19)[:, None]).astype(jnp.float32)
        h0 = (oh0 @ t1) + (oh1 @ t2) + pb_ref[...][None, :]
        dinv = lax.rsqrt(deg + 1.0)
        return (h0 @ w1) * dinv[:, None], dinv

    ye, de = node_y(xq[0], xq[2], jnp.sum(dege_ref[...], axis=0))
    yo, do = node_y(xq[1], xq[3], jnp.sum(dego_ref[...], axis=0))
    y_ref[...] = jnp.concatenate([ye, yo], axis=1)
    dinv_ref[...] = jnp.stack([de, do])


def _embed_call(xq, deg_parts, emb_table, depth_table, proj_W, proj_b, g1_W):
    return pl.pallas_call(
        _embed_body,
        grid=(GRID,),
        in_specs=[
            pl.BlockSpec((4, MBLK), lambda i: (0, i)),
            pl.BlockSpec((NC, MBLK), lambda i: (0, i)),
            pl.BlockSpec((NC, MBLK), lambda i: (0, i + GRID)),
            pl.BlockSpec((9, 16), lambda i: (0, 0)),
            pl.BlockSpec((20, 16), lambda i: (0, 0)),
            pl.BlockSpec((32, 64), lambda i: (0, 0)),
            pl.BlockSpec((64,), lambda i: (0,)),
            pl.BlockSpec((64, 64), lambda i: (0, 0)),
        ],
        out_specs=[
            pl.BlockSpec((MBLK, 128), lambda i: (i, 0)),
            pl.BlockSpec((2, MBLK), lambda i: (0, i)),
        ],
        out_shape=[
            jax.ShapeDtypeStruct((M, 128), jnp.float32),
            jax.ShapeDtypeStruct((2, M), jnp.float32),
        ],
    )(xq, deg_parts, deg_parts, emb_table, depth_table, proj_W, proj_b, g1_W)


def _pair_scale(dinv_ref):
    dv = dinv_ref[...]
    de = jnp.broadcast_to(dv[0][:, None], (MBLK, 64))
    do = jnp.broadcast_to(dv[1][:, None], (MBLK, 64))
    return jnp.concatenate([de, do], axis=1)


def _blockdiag(w):
    z = jnp.zeros((64, 64), jnp.float32)
    return jnp.concatenate([jnp.concatenate([w, z], axis=1),
                            jnp.concatenate([z, w], axis=1)], axis=0)


def _post1_body(a_ref, y_ref, dinv_ref, b_ref, w_ref, z_ref):
    scale = _pair_scale(dinv_ref)
    b2 = jnp.concatenate([b_ref[...], b_ref[...]])
    h = (a_ref[...] + y_ref[...]) * scale + b2[None, :]
    h = jnp.maximum(h, 0.0)
    z_ref[...] = (h @ _blockdiag(w_ref[...])) * scale


def _post1_call(a, y, dinvT, b1, g2_W):
    return pl.pallas_call(
        _post1_body,
        grid=(GRID,),
        in_specs=[
            pl.BlockSpec((MBLK, 128), lambda i: (i, 0)),
            pl.BlockSpec((MBLK, 128), lambda i: (i, 0)),
            pl.BlockSpec((2, MBLK), lambda i: (0, i)),
            pl.BlockSpec((64,), lambda i: (0,)),
            pl.BlockSpec((64, 64), lambda i: (0, 0)),
        ],
        out_specs=pl.BlockSpec((MBLK, 128), lambda i: (i, 0)),
        out_shape=jax.ShapeDtypeStruct((M, 128), jnp.float32),
    )(a, y, dinvT, b1, g2_W)


def _post2_body(a_ref, y_ref, dinv_ref, b_ref, batch_ref, fw_ref, fb_ref,
                out_ref, sacc, cacc):
    i = pl.program_id(0)

    @pl.when(i == 0)
    def _():
        sacc[...] = jnp.zeros_like(sacc)
        cacc[...] = jnp.zeros_like(cacc)

    scale = _pair_scale(dinv_ref)
    b2 = jnp.concatenate([b_ref[...], b_ref[...]])
    h = (a_ref[...] + y_ref[...]) * scale + b2[None, :]
    h = jnp.maximum(h, 0.0)
    bt = batch_ref[...]
    ohTe = (lax.broadcasted_iota(jnp.int32, (G, MBLK), 0)
            == bt[0][None, :]).astype(jnp.float32)
    ohTo = (lax.broadcasted_iota(jnp.int32, (G, MBLK), 0)
            == bt[1][None, :]).astype(jnp.float32)
    sacc[...] += ohTe @ h[:, :64] + ohTo @ h[:, 64:]
    cacc[...] += jnp.sum(ohTe, axis=1) + jnp.sum(ohTo, axis=1)
    pooled = sacc[...] / jnp.maximum(cacc[...], 1.0)[:, None]
    out_ref[...] = pooled @ fw_ref[...] + fb_ref[...][None, :]


def _post2_call(a, y, dinvT, b2, batchT, final_W, final_b):
    return pl.pallas_call(
        _post2_body,
        grid=(GRID,),
        in_specs=[
            pl.BlockSpec((MBLK, 128), lambda i: (i, 0)),
            pl.BlockSpec((MBLK, 128), lambda i: (i, 0)),
            pl.BlockSpec((2, MBLK), lambda i: (0, i)),
            pl.BlockSpec((64,), lambda i: (0,)),
            pl.BlockSpec((2, MBLK), lambda i: (0, i)),
            pl.BlockSpec((64, 128), lambda i: (0, 0)),
            pl.BlockSpec((128,), lambda i: (0,)),
        ],
        out_specs=pl.BlockSpec((G, 128), lambda i: (0, 0)),
        out_shape=jax.ShapeDtypeStruct((G, 128), jnp.float32),
        scratch_shapes=[
            pltpu.VMEM((G, 64), jnp.float32),
            pltpu.VMEM((G,), jnp.float32),
        ],
    )(a, y, dinvT, b2, batchT, final_W, final_b)


def kernel(x, edge_index, batch, emb_table, depth_table, proj_W, proj_b,
           g1_W, g1_b, g2_W, g2_b, final_W, final_b):
    xT = jnp.pad(x.T, ((0, 0), (0, N_PAD - N_NODES)))
    xq = xT.reshape(2, M, 2).transpose(0, 2, 1).reshape(4, M)
    batchT = jnp.pad(batch, (0, N_PAD - N_NODES),
                     constant_values=G).reshape(M, 2).T
    zeros_sl = jnp.zeros((SLICE, 32), jnp.float32)

    deg_parts = _deg_kernel(edge_index).reshape(NC, N_PAD)
    y, dinvT = _embed_call(xq, deg_parts, emb_table, depth_table,
                           proj_W, proj_b, g1_W)
    a = _conv_kernel(y.reshape(2 * N_PAD, 32), edge_index, zeros_sl)
    z = _post1_call(a.reshape(M, 128), y, dinvT, g1_b, g2_W)
    a2 = _conv_kernel(z.reshape(2 * N_PAD, 32), edge_index, zeros_sl)
    return _post2_call(a2.reshape(M, 128), z, dinvT, g2_b, batchT,
                       final_W, final_b)
